# Initial kernel scaffold; baseline (speedup 1.0000x reference)
import jax
import jax.numpy as jnp
from jax.experimental import pallas as pl


def kernel(x, llist, b0_w_t1, b0_b_t1, b0_a_t1, b0_theta, b0_b_s, b0_w_t2, b0_b_t2, b0_a_t2, b0_ln_g, b0_ln_b, b1_w_t1, b1_b_t1, b1_a_t1, b1_theta, b1_b_s, b1_w_t2, b1_b_t2, b1_a_t2, b1_ln_g, b1_ln_b, out_w_t, out_b_t, out_a_t, out_ln_g, out_ln_b, out_w_s, out_b_s, out_w_fc, out_b_fc):
    raise NotImplementedError("write your pallas kernel here")


# speedup vs baseline: 1.1329x; 1.1329x over previous
import functools

import jax
import jax.numpy as jnp
from jax import lax
from jax.experimental import pallas as pl
from jax.experimental.pallas import tpu as pltpu

_BB = 4
_N = 256
_KT = 3


def _glu(x_flat, n, t_in, kt, w, b, a):
    co = w.shape[2] // 2
    to = t_in - kt + 1
    acc = jnp.dot(x_flat[0:to * n, :], w[0], preferred_element_type=jnp.float32)
    for k in range(1, kt):
        acc = acc + jnp.dot(x_flat[k * n:(k + to) * n, :], w[k],
                            preferred_element_type=jnp.float32)
    acc = acc + b
    res = jnp.dot(x_flat[(kt - 1) * n:(kt - 1 + to) * n, :], a,
                  preferred_element_type=jnp.float32)
    return (acc[:, :co] + res) * jax.nn.sigmoid(acc[:, co:])


def _cheb(x1, llist, theta, bs, n, to1):
    ks = llist.shape[0]
    ct = x1.shape[1]
    x1_3 = x1.reshape(to1, n, ct)
    acc = None
    for k in range(ks):
        lk = jnp.broadcast_to(llist[k], (to1, n, n))
        lx = lax.dot_general(lk, x1_3, (((2,), (1,)), ((0,), (0,))),
                             preferred_element_type=jnp.float32)
        contrib = jnp.dot(lx.reshape(to1 * n, ct), theta[k],
                          preferred_element_type=jnp.float32)
        acc = contrib if acc is None else acc + contrib
    return jax.nn.relu(acc + bs)


def _layernorm_tn(x_flat, to, n, c, g, b):
    x3 = x_flat.reshape(to, n, c)
    mu = jnp.mean(x3, axis=(1, 2), keepdims=True)
    var = jnp.mean((x3 - mu) ** 2, axis=(1, 2), keepdims=True)
    return (x3 - mu) * lax.rsqrt(var + 1e-5) * g + b


def _stgcn_kernel(x_ref, l_ref,
                  w1a_ref, b1a_ref, a1a_ref, th0_ref, bs0_ref,
                  w2a_ref, b2a_ref, a2a_ref, g0_ref, be0_ref,
                  w1b_ref, b1b_ref, a1b_ref, th1_ref, bs1_ref,
                  w2b_ref, b2b_ref, a2b_ref, g1_ref, be1_ref,
                  wt_ref, bt_ref, at_ref, gh_ref, beh_ref,
                  ws_ref, bso_ref, wf_ref, bf_ref,
                  o_ref, f0_ref, f1_ref):
    n = _N
    kt = _KT
    llist = l_ref[...]
    for bi in range(_BB):
        t_in = x_ref.shape[1]
        x = x_ref[bi].reshape(t_in * n, x_ref.shape[3])

        to1 = t_in - kt + 1
        to2 = to1 - kt + 1
        x1 = _glu(x, n, t_in, kt, w1a_ref[...], b1a_ref[...], a1a_ref[...])
        x2 = _cheb(x1, llist, th0_ref[...], bs0_ref[...], n, to1)
        f0_ref[bi] = x2.reshape(to1, n, x2.shape[1])
        x3 = _glu(x2, n, to1, kt, w2a_ref[...], b2a_ref[...], a2a_ref[...])
        xa = _layernorm_tn(x3, to2, n, x3.shape[1], g0_ref[...], be0_ref[...])

        t2 = to2
        to3 = t2 - kt + 1
        to4 = to3 - kt + 1
        xf = xa.reshape(t2 * n, xa.shape[2])
        y1 = _glu(xf, n, t2, kt, w1b_ref[...], b1b_ref[...], a1b_ref[...])
        y2 = _cheb(y1, llist, th1_ref[...], bs1_ref[...], n, to3)
        f1_ref[bi] = y2.reshape(to3, n, y2.shape[1])
        y3 = _glu(y2, n, to3, kt, w2b_ref[...], b2b_ref[...], a2b_ref[...])
        yb = _layernorm_tn(y3, to4, n, y3.shape[1], g1_ref[...], be1_ref[...])

        ko = to4
        zf = yb.reshape(ko * n, yb.shape[2])
        z1 = _glu(zf, n, ko, ko, wt_ref[...], bt_ref[...], at_ref[...])
        mu = jnp.mean(z1)
        var = jnp.mean((z1 - mu) ** 2)
        z2 = (z1 - mu) * lax.rsqrt(var + 1e-5) * gh_ref[...] + beh_ref[...]
        s = jax.nn.sigmoid(jnp.dot(z2, ws_ref[...],
                                   preferred_element_type=jnp.float32)
                           + bso_ref[...])
        o_ref[bi] = jnp.dot(s, wf_ref[...],
                            preferred_element_type=jnp.float32) + bf_ref[...]


def kernel(x, llist, b0_w_t1, b0_b_t1, b0_a_t1, b0_theta, b0_b_s, b0_w_t2,
           b0_b_t2, b0_a_t2, b0_ln_g, b0_ln_b, b1_w_t1, b1_b_t1, b1_a_t1,
           b1_theta, b1_b_s, b1_w_t2, b1_b_t2, b1_a_t2, b1_ln_g, b1_ln_b,
           out_w_t, out_b_t, out_a_t, out_ln_g, out_ln_b, out_w_s, out_b_s,
           out_w_fc, out_b_fc):
    bsz, t_in, n, cin = x.shape
    kt = _KT
    to1 = t_in - kt + 1
    to2 = to1 - kt + 1
    to3 = to2 - kt + 1
    cs0 = b0_theta.shape[2]
    cs1 = b1_theta.shape[2]
    full = lambda shape: pl.BlockSpec(shape, lambda b, _s=shape: (0,) * len(_s))
    weights = (llist, b0_w_t1, b0_b_t1, b0_a_t1, b0_theta, b0_b_s, b0_w_t2,
               b0_b_t2, b0_a_t2, b0_ln_g, b0_ln_b, b1_w_t1, b1_b_t1, b1_a_t1,
               b1_theta, b1_b_s, b1_w_t2, b1_b_t2, b1_a_t2, b1_ln_g, b1_ln_b,
               out_w_t, out_b_t, out_a_t, out_ln_g, out_ln_b, out_w_s,
               out_b_s, out_w_fc, out_b_fc)
    out, f0, f1 = pl.pallas_call(
        _stgcn_kernel,
        out_shape=(jax.ShapeDtypeStruct((bsz, n, 1), jnp.float32),
                   jax.ShapeDtypeStruct((bsz, to1, n, cs0), jnp.float32),
                   jax.ShapeDtypeStruct((bsz, to3, n, cs1), jnp.float32)),
        grid=(bsz // _BB,),
        in_specs=[pl.BlockSpec((_BB, t_in, n, cin), lambda b: (b, 0, 0, 0))] +
                 [full(w.shape) for w in weights],
        out_specs=(pl.BlockSpec((_BB, n, 1), lambda b: (b, 0, 0)),
                   pl.BlockSpec((_BB, to1, n, cs0), lambda b: (b, 0, 0, 0)),
                   pl.BlockSpec((_BB, to3, n, cs1), lambda b: (b, 0, 0, 0))),
        compiler_params=pltpu.CompilerParams(
            dimension_semantics=("parallel",)),
    )(x, *weights)
    return out.reshape(bsz, 1, n, 1), [f0, f1]


# speedup vs baseline: 1.5403x; 1.3596x over previous
import jax
import jax.numpy as jnp
from jax import lax
from jax.experimental import pallas as pl
from jax.experimental.pallas import tpu as pltpu

_BB = 4
_N = 256
_KT = 3


def _glu_c(a, n, t_in, kt, w_stk_t, b_t, a_t):
    cin = a.shape[0]
    co = w_stk_t.shape[0] // 2
    to = t_in - kt + 1
    if cin % 8 == 0:
        a_stk = jnp.concatenate([a[:, k * n:(k + to) * n] for k in range(kt)],
                                axis=0)
        acc = jnp.dot(w_stk_t, a_stk, preferred_element_type=jnp.float32)
    else:
        acc = jnp.dot(w_stk_t[:, 0:cin], a[:, 0:to * n],
                      preferred_element_type=jnp.float32)
        for k in range(1, kt):
            acc = acc + jnp.dot(w_stk_t[:, k * cin:(k + 1) * cin],
                                a[:, k * n:(k + to) * n],
                                preferred_element_type=jnp.float32)
    acc = acc + b_t
    res = jnp.dot(a_t, a[:, (kt - 1) * n:(kt - 1 + to) * n],
                  preferred_element_type=jnp.float32)
    return (acc[:co] + res) * jax.nn.sigmoid(acc[co:])


def _cheb_c(h, l_cat, th_t, bs_t, n, to1):
    ks = l_cat.shape[1] // n
    ct = h.shape[0]
    outs = []
    for t in range(to1):
        g = jnp.dot(h[:, t * n:(t + 1) * n], l_cat,
                    preferred_element_type=jnp.float32)
        g_stk = jnp.concatenate([g[:, k * n:(k + 1) * n] for k in range(ks)],
                                axis=0)
        outs.append(jnp.dot(th_t, g_stk,
                            preferred_element_type=jnp.float32))
    x2 = jnp.concatenate(outs, axis=1)
    return jax.nn.relu(x2 + bs_t)


def _ln_c(x, g_t, b_t, seg_d, seg_u, n, to):
    s1 = jnp.sum(x, axis=0, keepdims=True)
    s2 = jnp.sum(x * x, axis=0, keepdims=True)
    mu_t = jnp.dot(s1, seg_d, preferred_element_type=jnp.float32)
    e2_t = jnp.dot(s2, seg_d, preferred_element_type=jnp.float32)
    rs_t = lax.rsqrt(e2_t - mu_t * mu_t + 1e-5)
    mu = jnp.dot(mu_t, seg_u, preferred_element_type=jnp.float32)
    rs = jnp.dot(rs_t, seg_u, preferred_element_type=jnp.float32)
    g_full = jnp.concatenate([g_t] * to, axis=1)
    b_full = jnp.concatenate([b_t] * to, axis=1)
    return (x - mu) * rs * g_full + b_full


def _stgcn_kernel(x_ref, lcat_ref,
                  w1a_ref, b1a_ref, a1a_ref, th0_ref, bs0_ref,
                  w2a_ref, b2a_ref, a2a_ref, g0_ref, be0_ref,
                  w1b_ref, b1b_ref, a1b_ref, th1_ref, bs1_ref,
                  w2b_ref, b2b_ref, a2b_ref, g1_ref, be1_ref,
                  wt_ref, bt_ref, at_ref, gh_ref, beh_ref,
                  ws_ref, bso_ref, wf_ref, bfc_ref,
                  sd0_ref, su0_ref, sd1_ref, su1_ref,
                  o_ref, f0_ref, f1_ref):
    n = _N
    kt = _KT
    l_cat = lcat_ref[...]
    for bi in range(_BB):
        t_in = x_ref.shape[2] // n
        a0 = x_ref[bi]

        to1 = t_in - kt + 1
        to2 = to1 - kt + 1
        h0 = _glu_c(a0, n, t_in, kt, w1a_ref[...], b1a_ref[...], a1a_ref[...])
        x2 = _cheb_c(h0, l_cat, th0_ref[...], bs0_ref[...], n, to1)
        f0_ref[bi] = x2
        h1 = _glu_c(x2, n, to1, kt, w2a_ref[...], b2a_ref[...], a2a_ref[...])
        a2 = _ln_c(h1, g0_ref[...], be0_ref[...], sd0_ref[...], su0_ref[...],
                   n, to2)

        to3 = to2 - kt + 1
        to4 = to3 - kt + 1
        h2 = _glu_c(a2, n, to2, kt, w1b_ref[...], b1b_ref[...], a1b_ref[...])
        y2 = _cheb_c(h2, l_cat, th1_ref[...], bs1_ref[...], n, to3)
        f1_ref[bi] = y2
        h3 = _glu_c(y2, n, to3, kt, w2b_ref[...], b2b_ref[...], a2b_ref[...])
        a4 = _ln_c(h3, g1_ref[...], be1_ref[...], sd1_ref[...], su1_ref[...],
                   n, to4)

        ko = to4
        z1 = _glu_c(a4, n, ko, ko, wt_ref[...], bt_ref[...], at_ref[...])
        mu = jnp.mean(z1)
        var = jnp.mean((z1 - mu) ** 2)
        z2 = (z1 - mu) * lax.rsqrt(var + 1e-5) * gh_ref[...] + beh_ref[...]
        s = jax.nn.sigmoid(jnp.dot(ws_ref[...], z2,
                                   preferred_element_type=jnp.float32)
                           + bso_ref[...])
        o_ref[bi] = jnp.dot(wf_ref[...], s,
                            preferred_element_type=jnp.float32) + bfc_ref[...]


def _tap_stack_t(w):
    kt, cin, co2 = w.shape
    return jnp.transpose(w, (2, 0, 1)).reshape(co2, kt * cin)


def _seg_mats(to, n, c):
    t_of = jnp.arange(to * n, dtype=jnp.int32) // n
    seg = (t_of[:, None] == jnp.arange(to, dtype=jnp.int32)[None, :])
    seg_d = seg.astype(jnp.float32) / float(c * n)
    seg_u = jnp.transpose(seg).astype(jnp.float32)
    return seg_d, seg_u


def kernel(x, llist, b0_w_t1, b0_b_t1, b0_a_t1, b0_theta, b0_b_s, b0_w_t2,
           b0_b_t2, b0_a_t2, b0_ln_g, b0_ln_b, b1_w_t1, b1_b_t1, b1_a_t1,
           b1_theta, b1_b_s, b1_w_t2, b1_b_t2, b1_a_t2, b1_ln_g, b1_ln_b,
           out_w_t, out_b_t, out_a_t, out_ln_g, out_ln_b, out_w_s, out_b_s,
           out_w_fc, out_b_fc):
    bsz, t_in, n, cin = x.shape
    kt = _KT
    to1 = t_in - kt + 1
    to2 = to1 - kt + 1
    to3 = to2 - kt + 1
    to4 = to3 - kt + 1
    cs0 = b0_theta.shape[2]
    cs1 = b1_theta.shape[2]
    c0 = b0_w_t2.shape[2] // 2
    c1 = b1_w_t2.shape[2] // 2

    x_t = jnp.transpose(x, (0, 3, 1, 2)).reshape(bsz, cin, t_in * n)
    l_cat = jnp.transpose(llist, (2, 0, 1)).reshape(n, llist.shape[0] * n)
    sd0, su0 = _seg_mats(to2, n, c0)
    sd1, su1 = _seg_mats(to4, n, c1)
    ops = (l_cat,
           _tap_stack_t(b0_w_t1), b0_b_t1.T, b0_a_t1.T,
           _tap_stack_t(b0_theta), b0_b_s.T,
           _tap_stack_t(b0_w_t2), b0_b_t2.T, b0_a_t2.T,
           b0_ln_g.T, b0_ln_b.T,
           _tap_stack_t(b1_w_t1), b1_b_t1.T, b1_a_t1.T,
           _tap_stack_t(b1_theta), b1_b_s.T,
           _tap_stack_t(b1_w_t2), b1_b_t2.T, b1_a_t2.T,
           b1_ln_g.T, b1_ln_b.T,
           _tap_stack_t(out_w_t), out_b_t.T, out_a_t.T,
           out_ln_g.T, out_ln_b.T,
           out_w_s.T, out_b_s.T, out_w_fc.T, out_b_fc.T,
           sd0, su0, sd1, su1)

    full = lambda shape: pl.BlockSpec(shape, lambda b, _s=shape: (0,) * len(_s))
    o, f0c, f1c = pl.pallas_call(
        _stgcn_kernel,
        out_shape=(jax.ShapeDtypeStruct((bsz, 1, n), jnp.float32),
                   jax.ShapeDtypeStruct((bsz, cs0, to1 * n), jnp.float32),
                   jax.ShapeDtypeStruct((bsz, cs1, to3 * n), jnp.float32)),
        grid=(bsz // _BB,),
        in_specs=[pl.BlockSpec((_BB, cin, t_in * n), lambda b: (b, 0, 0))] +
                 [full(w.shape) for w in ops],
        out_specs=(pl.BlockSpec((_BB, 1, n), lambda b: (b, 0, 0)),
                   pl.BlockSpec((_BB, cs0, to1 * n), lambda b: (b, 0, 0)),
                   pl.BlockSpec((_BB, cs1, to3 * n), lambda b: (b, 0, 0))),
        compiler_params=pltpu.CompilerParams(
            dimension_semantics=("parallel",)),
    )(x_t, *ops)
    f0 = jnp.transpose(f0c.reshape(bsz, cs0, to1, n), (0, 2, 3, 1))
    f1 = jnp.transpose(f1c.reshape(bsz, cs1, to3, n), (0, 2, 3, 1))
    return o.reshape(bsz, 1, n, 1), [f0, f1]


# speedup vs baseline: 1.6759x; 1.0881x over previous
import jax
import jax.numpy as jnp
from jax import lax
from jax.experimental import pallas as pl
from jax.experimental.pallas import tpu as pltpu

_BB = 8
_N = 256
_KT = 3


def _glu_c(a, n, t_in, kt, w_stk_t, b_t, a_t):
    cin = a.shape[0]
    co = w_stk_t.shape[0] // 2
    to = t_in - kt + 1
    if cin % 8 == 0:
        a_stk = jnp.concatenate([a[:, k * n:(k + to) * n] for k in range(kt)],
                                axis=0)
        acc = jnp.dot(w_stk_t, a_stk, preferred_element_type=jnp.float32)
    else:
        acc = jnp.dot(w_stk_t[:, 0:cin], a[:, 0:to * n],
                      preferred_element_type=jnp.float32)
        for k in range(1, kt):
            acc = acc + jnp.dot(w_stk_t[:, k * cin:(k + 1) * cin],
                                a[:, k * n:(k + to) * n],
                                preferred_element_type=jnp.float32)
    acc = acc + b_t
    res = jnp.dot(a_t, a[:, (kt - 1) * n:(kt - 1 + to) * n],
                  preferred_element_type=jnp.float32)
    return (acc[:co] + res) * jax.nn.sigmoid(acc[co:])


def _cheb_multi(hs, l_cat, th_t, bs_t, n, to1):
    ks = l_cat.shape[1] // n
    outs = [[] for _ in hs]
    for t in range(to1):
        for bi, h in enumerate(hs):
            g = jnp.dot(h[:, t * n:(t + 1) * n], l_cat,
                        preferred_element_type=jnp.float32)
            g_stk = jnp.concatenate(
                [g[:, k * n:(k + 1) * n] for k in range(ks)], axis=0)
            outs[bi].append(jnp.dot(th_t, g_stk,
                                    preferred_element_type=jnp.float32))
    return [jax.nn.relu(jnp.concatenate(o, axis=1) + bs_t) for o in outs]


def _ln_c(x, g_t, b_t, seg_d, seg_u, n, to):
    s1 = jnp.sum(x, axis=0, keepdims=True)
    mu_t = jnp.dot(s1, seg_d, preferred_element_type=jnp.float32)
    mu = jnp.dot(mu_t, seg_u, preferred_element_type=jnp.float32)
    xc = x - mu
    s2 = jnp.sum(xc * xc, axis=0, keepdims=True)
    var_t = jnp.dot(s2, seg_d, preferred_element_type=jnp.float32)
    rs_t = lax.rsqrt(var_t + 1e-5)
    rs = jnp.dot(rs_t, seg_u, preferred_element_type=jnp.float32)
    g_full = jnp.concatenate([g_t] * to, axis=1)
    b_full = jnp.concatenate([b_t] * to, axis=1)
    return xc * rs * g_full + b_full


def _stgcn_kernel(x_ref, lcat_ref,
                  w1a_ref, b1a_ref, a1a_ref, th0_ref, bs0_ref,
                  w2a_ref, b2a_ref, a2a_ref, g0_ref, be0_ref,
                  w1b_ref, b1b_ref, a1b_ref, th1_ref, bs1_ref,
                  w2b_ref, b2b_ref, a2b_ref, g1_ref, be1_ref,
                  wt_ref, bt_ref, at_ref, gh_ref, beh_ref,
                  ws_ref, bso_ref, wf_ref, bfc_ref,
                  sd0_ref, su0_ref, sd1_ref, su1_ref,
                  o_ref, f0_ref, f1_ref):
    n = _N
    kt = _KT
    l_cat = lcat_ref[...]
    t_in = x_ref.shape[2] // n
    to1 = t_in - kt + 1
    to2 = to1 - kt + 1
    to3 = to2 - kt + 1
    to4 = to3 - kt + 1
    bb = range(_BB)

    h0 = [_glu_c(x_ref[bi], n, t_in, kt, w1a_ref[...], b1a_ref[...],
                 a1a_ref[...]) for bi in bb]
    x2 = _cheb_multi(h0, l_cat, th0_ref[...], bs0_ref[...], n, to1)
    for bi in bb:
        f0_ref[bi] = x2[bi]
    h1 = [_glu_c(x2[bi], n, to1, kt, w2a_ref[...], b2a_ref[...],
                 a2a_ref[...]) for bi in bb]
    a2 = [_ln_c(h1[bi], g0_ref[...], be0_ref[...], sd0_ref[...],
                su0_ref[...], n, to2) for bi in bb]

    h2 = [_glu_c(a2[bi], n, to2, kt, w1b_ref[...], b1b_ref[...],
                 a1b_ref[...]) for bi in bb]
    y2 = _cheb_multi(h2, l_cat, th1_ref[...], bs1_ref[...], n, to3)
    for bi in bb:
        f1_ref[bi] = y2[bi]
    h3 = [_glu_c(y2[bi], n, to3, kt, w2b_ref[...], b2b_ref[...],
                 a2b_ref[...]) for bi in bb]
    a4 = [_ln_c(h3[bi], g1_ref[...], be1_ref[...], sd1_ref[...],
                su1_ref[...], n, to4) for bi in bb]

    ko = to4
    for bi in bb:
        z1 = _glu_c(a4[bi], n, ko, ko, wt_ref[...], bt_ref[...], at_ref[...])
        mu = jnp.mean(z1)
        var = jnp.mean((z1 - mu) ** 2)
        z2 = (z1 - mu) * lax.rsqrt(var + 1e-5) * gh_ref[...] + beh_ref[...]
        s = jax.nn.sigmoid(jnp.dot(ws_ref[...], z2,
                                   preferred_element_type=jnp.float32)
                           + bso_ref[...])
        o_ref[bi] = jnp.dot(wf_ref[...], s,
                            preferred_element_type=jnp.float32) + bfc_ref[...]


def _tap_stack_t(w):
    kt, cin, co2 = w.shape
    return jnp.transpose(w, (2, 0, 1)).reshape(co2, kt * cin)


def _seg_mats(to, n, c):
    t_of = jnp.arange(to * n, dtype=jnp.int32) // n
    seg = (t_of[:, None] == jnp.arange(to, dtype=jnp.int32)[None, :])
    seg_d = seg.astype(jnp.float32) / float(c * n)
    seg_u = jnp.transpose(seg).astype(jnp.float32)
    return seg_d, seg_u


def kernel(x, llist, b0_w_t1, b0_b_t1, b0_a_t1, b0_theta, b0_b_s, b0_w_t2,
           b0_b_t2, b0_a_t2, b0_ln_g, b0_ln_b, b1_w_t1, b1_b_t1, b1_a_t1,
           b1_theta, b1_b_s, b1_w_t2, b1_b_t2, b1_a_t2, b1_ln_g, b1_ln_b,
           out_w_t, out_b_t, out_a_t, out_ln_g, out_ln_b, out_w_s, out_b_s,
           out_w_fc, out_b_fc):
    bsz, t_in, n, cin = x.shape
    kt = _KT
    to1 = t_in - kt + 1
    to2 = to1 - kt + 1
    to3 = to2 - kt + 1
    to4 = to3 - kt + 1
    cs0 = b0_theta.shape[2]
    cs1 = b1_theta.shape[2]
    c0 = b0_w_t2.shape[2] // 2
    c1 = b1_w_t2.shape[2] // 2

    x_t = jnp.transpose(x, (0, 3, 1, 2)).reshape(bsz, cin, t_in * n)
    l_cat = jnp.transpose(llist, (2, 0, 1)).reshape(n, llist.shape[0] * n)
    sd0, su0 = _seg_mats(to2, n, c0)
    sd1, su1 = _seg_mats(to4, n, c1)
    ops = (l_cat,
           _tap_stack_t(b0_w_t1), b0_b_t1.T, b0_a_t1.T,
           _tap_stack_t(b0_theta), b0_b_s.T,
           _tap_stack_t(b0_w_t2), b0_b_t2.T, b0_a_t2.T,
           b0_ln_g.T, b0_ln_b.T,
           _tap_stack_t(b1_w_t1), b1_b_t1.T, b1_a_t1.T,
           _tap_stack_t(b1_theta), b1_b_s.T,
           _tap_stack_t(b1_w_t2), b1_b_t2.T, b1_a_t2.T,
           b1_ln_g.T, b1_ln_b.T,
           _tap_stack_t(out_w_t), out_b_t.T, out_a_t.T,
           out_ln_g.T, out_ln_b.T,
           out_w_s.T, out_b_s.T, out_w_fc.T, out_b_fc.T,
           sd0, su0, sd1, su1)

    full = lambda shape: pl.BlockSpec(shape, lambda b, _s=shape: (0,) * len(_s))
    o, f0c, f1c = pl.pallas_call(
        _stgcn_kernel,
        out_shape=(jax.ShapeDtypeStruct((bsz, 1, n), jnp.float32),
                   jax.ShapeDtypeStruct((bsz, cs0, to1 * n), jnp.float32),
                   jax.ShapeDtypeStruct((bsz, cs1, to3 * n), jnp.float32)),
        grid=(bsz // _BB,),
        in_specs=[pl.BlockSpec((_BB, cin, t_in * n), lambda b: (b, 0, 0))] +
                 [full(w.shape) for w in ops],
        out_specs=(pl.BlockSpec((_BB, 1, n), lambda b: (b, 0, 0)),
                   pl.BlockSpec((_BB, cs0, to1 * n), lambda b: (b, 0, 0)),
                   pl.BlockSpec((_BB, cs1, to3 * n), lambda b: (b, 0, 0))),
        compiler_params=pltpu.CompilerParams(
            dimension_semantics=("parallel",)),
    )(x_t, *ops)
    f0 = jnp.transpose(f0c.reshape(bsz, cs0, to1, n), (0, 2, 3, 1))
    f1 = jnp.transpose(f1c.reshape(bsz, cs1, to3, n), (0, 2, 3, 1))
    return o.reshape(bsz, 1, n, 1), [f0, f1]


# speedup vs baseline: 1.8968x; 1.1318x over previous
import jax
import jax.numpy as jnp
from jax import lax
from jax.experimental import pallas as pl
from jax.experimental.pallas import tpu as pltpu

_BB = 8
_N = 256
_KT = 3


def _glu_c(a, n, t_in, kt, w_stk_t, b_t, a_t):
    cin = a.shape[0]
    co = w_stk_t.shape[0] // 2
    to = t_in - kt + 1
    if cin % 8 == 0:
        a_stk = jnp.concatenate([a[:, k * n:(k + to) * n] for k in range(kt)],
                                axis=0)
        acc = jnp.dot(w_stk_t, a_stk, preferred_element_type=jnp.float32)
    else:
        acc = jnp.dot(w_stk_t[:, 0:cin], a[:, 0:to * n],
                      preferred_element_type=jnp.float32)
        for k in range(1, kt):
            acc = acc + jnp.dot(w_stk_t[:, k * cin:(k + 1) * cin],
                                a[:, k * n:(k + to) * n],
                                preferred_element_type=jnp.float32)
    acc = acc + b_t
    res = jnp.dot(a_t, a[:, (kt - 1) * n:(kt - 1 + to) * n],
                  preferred_element_type=jnp.float32)
    return (acc[:co] + res) * jax.nn.sigmoid(acc[co:])


def _cheb_multi(hs, l_cat, th_t, bs_t, n, to1):
    ks = l_cat.shape[1] // n
    outs = [[] for _ in hs]
    for t in range(to1):
        for bi, h in enumerate(hs):
            g = jnp.dot(h[:, t * n:(t + 1) * n], l_cat,
                        preferred_element_type=jnp.float32)
            g_stk = jnp.concatenate(
                [g[:, k * n:(k + 1) * n] for k in range(ks)], axis=0)
            outs[bi].append(jnp.dot(th_t, g_stk,
                                    preferred_element_type=jnp.float32))
    return [jax.nn.relu(jnp.concatenate(o, axis=1) + bs_t) for o in outs]


def _seg_bsum(row, n, to):
    r = row.reshape(to, n)
    s = jnp.sum(r, axis=1, keepdims=True)
    return jnp.broadcast_to(s, (to, n)).reshape(1, to * n)


def _ln_c(x, g_t, b_t, n, to):
    c = x.shape[0]
    inv = 1.0 / float(c * n)
    s1 = jnp.sum(x, axis=0, keepdims=True)
    mu = _seg_bsum(s1, n, to) * inv
    xc = x - mu
    s2 = jnp.sum(xc * xc, axis=0, keepdims=True)
    var = _seg_bsum(s2, n, to) * inv
    rs = lax.rsqrt(var + 1e-5)
    g_full = jnp.concatenate([g_t] * to, axis=1)
    b_full = jnp.concatenate([b_t] * to, axis=1)
    return xc * rs * g_full + b_full


def _stgcn_kernel(x_ref, lcat_ref,
                  w1a_ref, b1a_ref, a1a_ref, th0_ref, bs0_ref,
                  w2a_ref, b2a_ref, a2a_ref, g0_ref, be0_ref,
                  w1b_ref, b1b_ref, a1b_ref, th1_ref, bs1_ref,
                  w2b_ref, b2b_ref, a2b_ref, g1_ref, be1_ref,
                  wt_ref, bt_ref, at_ref, gh_ref, beh_ref,
                  ws_ref, bso_ref, wf_ref, bfc_ref,
                  o_ref, f0_ref, f1_ref):
    n = _N
    kt = _KT
    l_cat = lcat_ref[...]
    t_in = x_ref.shape[2] // n
    to1 = t_in - kt + 1
    to2 = to1 - kt + 1
    to3 = to2 - kt + 1
    to4 = to3 - kt + 1
    bb = range(_BB)

    h0 = [_glu_c(x_ref[bi], n, t_in, kt, w1a_ref[...], b1a_ref[...],
                 a1a_ref[...]) for bi in bb]
    x2 = _cheb_multi(h0, l_cat, th0_ref[...], bs0_ref[...], n, to1)
    for bi in bb:
        f0_ref[bi] = x2[bi]
    h1 = [_glu_c(x2[bi], n, to1, kt, w2a_ref[...], b2a_ref[...],
                 a2a_ref[...]) for bi in bb]
    a2 = [_ln_c(h1[bi], g0_ref[...], be0_ref[...], n, to2) for bi in bb]

    h2 = [_glu_c(a2[bi], n, to2, kt, w1b_ref[...], b1b_ref[...],
                 a1b_ref[...]) for bi in bb]
    y2 = _cheb_multi(h2, l_cat, th1_ref[...], bs1_ref[...], n, to3)
    for bi in bb:
        f1_ref[bi] = y2[bi]
    h3 = [_glu_c(y2[bi], n, to3, kt, w2b_ref[...], b2b_ref[...],
                 a2b_ref[...]) for bi in bb]
    a4 = [_ln_c(h3[bi], g1_ref[...], be1_ref[...], n, to4) for bi in bb]

    ko = to4
    for bi in bb:
        z1 = _glu_c(a4[bi], n, ko, ko, wt_ref[...], bt_ref[...], at_ref[...])
        mu = jnp.mean(z1)
        var = jnp.mean((z1 - mu) ** 2)
        z2 = (z1 - mu) * lax.rsqrt(var + 1e-5) * gh_ref[...] + beh_ref[...]
        s = jax.nn.sigmoid(jnp.dot(ws_ref[...], z2,
                                   preferred_element_type=jnp.float32)
                           + bso_ref[...])
        o_ref[bi] = jnp.dot(wf_ref[...], s,
                            preferred_element_type=jnp.float32) + bfc_ref[...]


def _tap_stack_t(w):
    kt, cin, co2 = w.shape
    return jnp.transpose(w, (2, 0, 1)).reshape(co2, kt * cin)


def kernel(x, llist, b0_w_t1, b0_b_t1, b0_a_t1, b0_theta, b0_b_s, b0_w_t2,
           b0_b_t2, b0_a_t2, b0_ln_g, b0_ln_b, b1_w_t1, b1_b_t1, b1_a_t1,
           b1_theta, b1_b_s, b1_w_t2, b1_b_t2, b1_a_t2, b1_ln_g, b1_ln_b,
           out_w_t, out_b_t, out_a_t, out_ln_g, out_ln_b, out_w_s, out_b_s,
           out_w_fc, out_b_fc):
    bsz, t_in, n, cin = x.shape
    kt = _KT
    to1 = t_in - kt + 1
    to2 = to1 - kt + 1
    to3 = to2 - kt + 1
    to4 = to3 - kt + 1
    cs0 = b0_theta.shape[2]
    cs1 = b1_theta.shape[2]
    c0 = b0_w_t2.shape[2] // 2
    c1 = b1_w_t2.shape[2] // 2

    x_t = jnp.transpose(x, (0, 3, 1, 2)).reshape(bsz, cin, t_in * n)
    l_cat = jnp.transpose(llist, (2, 0, 1)).reshape(n, llist.shape[0] * n)
    ops = (l_cat,
           _tap_stack_t(b0_w_t1), b0_b_t1.T, b0_a_t1.T,
           _tap_stack_t(b0_theta), b0_b_s.T,
           _tap_stack_t(b0_w_t2), b0_b_t2.T, b0_a_t2.T,
           b0_ln_g.T, b0_ln_b.T,
           _tap_stack_t(b1_w_t1), b1_b_t1.T, b1_a_t1.T,
           _tap_stack_t(b1_theta), b1_b_s.T,
           _tap_stack_t(b1_w_t2), b1_b_t2.T, b1_a_t2.T,
           b1_ln_g.T, b1_ln_b.T,
           _tap_stack_t(out_w_t), out_b_t.T, out_a_t.T,
           out_ln_g.T, out_ln_b.T,
           out_w_s.T, out_b_s.T, out_w_fc.T, out_b_fc.T)

    full = lambda shape: pl.BlockSpec(shape, lambda b, _s=shape: (0,) * len(_s))
    o, f0c, f1c = pl.pallas_call(
        _stgcn_kernel,
        out_shape=(jax.ShapeDtypeStruct((bsz, 1, n), jnp.float32),
                   jax.ShapeDtypeStruct((bsz, cs0, to1 * n), jnp.float32),
                   jax.ShapeDtypeStruct((bsz, cs1, to3 * n), jnp.float32)),
        grid=(bsz // _BB,),
        in_specs=[pl.BlockSpec((_BB, cin, t_in * n), lambda b: (b, 0, 0))] +
                 [full(w.shape) for w in ops],
        out_specs=(pl.BlockSpec((_BB, 1, n), lambda b: (b, 0, 0)),
                   pl.BlockSpec((_BB, cs0, to1 * n), lambda b: (b, 0, 0)),
                   pl.BlockSpec((_BB, cs1, to3 * n), lambda b: (b, 0, 0))),
        compiler_params=pltpu.CompilerParams(
            dimension_semantics=("parallel",)),
    )(x_t, *ops)
    f0 = jnp.transpose(f0c.reshape(bsz, cs0, to1, n), (0, 2, 3, 1))
    f1 = jnp.transpose(f1c.reshape(bsz, cs1, to3, n), (0, 2, 3, 1))
    return o.reshape(bsz, 1, n, 1), [f0, f1]
